# Initial kernel scaffold; baseline (speedup 1.0000x reference)
import jax
import jax.numpy as jnp
from jax.experimental import pallas as pl


def kernel(x, edge_index, W_embed1, W_embed2, W_adapt, b_adapt, Wn1, Wn2, Wg1, Wg2, Wf1, bf1, Wf2, bf2):
    raise NotImplementedError("write your pallas kernel here")


# speedup vs baseline: 1.0000x; 1.0000x over previous
import jax
import jax.numpy as jnp
from jax.experimental import pallas as pl


def _gcn(h, src, dst, norm_src, norm_dst, W):
    n = h.shape[0]
    msg = h[src] * norm_src[src][:, None]
    agg = jnp.zeros((n, h.shape[1]), dtype=h.dtype).at[dst].add(msg)
    agg = agg * norm_dst[:, None]
    return agg @ W


def _layernorm(h):
    mu = h.mean(axis=-1, keepdims=True)
    var = h.var(axis=-1, keepdims=True)
    return (h - mu) / jnp.sqrt(var + 1e-5)


def _head_kernel(fused_ref, Wf1_ref, bf1_ref, Wf2_ref, bf2_ref, out_ref):
    fused = fused_ref[...]
    h = jnp.maximum(
        jax.lax.dot_general(fused, Wf1_ref[...], (((1,), (0,)), ((), ()))) + bf1_ref[...][None, :],
        0.0,
    )
    out_ref[...] = jax.lax.dot_general(h, Wf2_ref[...], (((1,), (0,)), ((), ()))) + bf2_ref[...][None, :]


def kernel(x, edge_index, W_embed1, W_embed2, W_adapt, b_adapt, Wn1, Wn2, Wg1, Wg2, Wf1, bf1, Wf2, bf2):
    src = edge_index[0]
    dst = edge_index[1]
    n = x.shape[0]
    deg_out = jnp.zeros((n,), dtype=x.dtype).at[src].add(1.0)
    deg_in = jnp.zeros((n,), dtype=x.dtype).at[dst].add(1.0)
    norm_src = 1.0 / jnp.sqrt(jnp.clip(deg_out, 1.0))
    norm_dst = 1.0 / jnp.sqrt(jnp.clip(deg_in, 1.0))

    h0 = jax.nn.relu(_gcn(x, src, dst, norm_src, norm_dst, W_embed1))
    global_h = _gcn(h0, src, dst, norm_src, norm_dst, W_embed2)
    adapted = global_h @ W_adapt + b_adapt

    def rho(h, W1, W2):
        h1 = jax.nn.relu(_gcn(h, src, dst, norm_src, norm_dst, W1)) + h
        h1 = _layernorm(h1)
        h2 = _gcn(h1, src, dst, norm_src, norm_dst, W2) + h1
        return h2

    hn = rho(adapted, Wn1, Wn2)
    hg = rho(adapted, Wg1, Wg2)
    rep_n = hn.mean(axis=0, keepdims=True)
    rep_g = hg.mean(axis=0, keepdims=True)
    fused = jnp.concatenate([rep_n, rep_g], axis=-1)

    logits = pl.pallas_call(
        _head_kernel,
        out_shape=jax.ShapeDtypeStruct((1, Wf2.shape[1]), x.dtype),
    )(fused, Wf1, bf1, Wf2, bf2)
    return logits


# speedup vs baseline: 5.1275x; 5.1273x over previous
import functools

import jax
import jax.numpy as jnp
from jax import lax
from jax.experimental import pallas as pl
from jax.experimental.pallas import tpu as pltpu
from jax.experimental.pallas import tpu_sc as plsc

_NC = 2
_NS = 16
_NW = _NC * _NS
_K = 128
_DEGW = 16
_NPAD = 10240
_RPN = _NPAD // _NS

_f32 = jnp.float32


def _mesh():
    return plsc.VectorSubcoreMesh(core_axis_name="c", subcore_axis_name="s")


def _sc_degree(src3, dst3, ones_blk, zeros_blk, d):
    cpw = src3.shape[2] // _K

    @functools.partial(
        pl.kernel,
        out_type=(
            jax.ShapeDtypeStruct((_NPAD, d), _f32),
            jax.ShapeDtypeStruct((_NPAD, d), _f32),
        ),
        mesh=_mesh(),
        scratch_types=[
            pltpu.VMEM((_K,), jnp.int32),
            pltpu.VMEM((_K, d), _f32),
            pltpu.VMEM_SHARED((_NPAD, d), _f32),
        ],
    )
    def deg_kernel(src_hbm, dst_hbm, ones_hbm, z_hbm, outs_hbm, outd_hbm,
                   sidx, ones_v, tab):
        c = lax.axis_index("c")
        s = lax.axis_index("s")
        base = pl.multiple_of(s * _RPN, 8)
        pltpu.sync_copy(z_hbm, tab.at[pl.ds(base, _RPN)])
        pltpu.sync_copy(ones_hbm, ones_v)
        plsc.subcore_barrier()

        def run(e_hbm):
            def outer(b, carry):
                blk = 2 * s + b

                def body(j, carry2):
                    off = pl.multiple_of(j * _K, 8)
                    pltpu.sync_copy(e_hbm.at[blk, 0, pl.ds(off, _K)], sidx)
                    pltpu.sync_copy(ones_v, tab.at[sidx], add=True)
                    return carry2

                lax.fori_loop(0, cpw, body, 0)
                return carry

            lax.fori_loop(0, 2, outer, 0)

        @pl.when(c == 0)
        def _():
            run(src_hbm)

        @pl.when(c == 1)
        def _():
            run(dst_hbm)

        plsc.subcore_barrier()

        @pl.when(c == 0)
        def _():
            pltpu.sync_copy(tab.at[pl.ds(base, _RPN)],
                            outs_hbm.at[pl.ds(base, _RPN)])

        @pl.when(c == 1)
        def _():
            pltpu.sync_copy(tab.at[pl.ds(base, _RPN)],
                            outd_hbm.at[pl.ds(base, _RPN)])

    return deg_kernel(src3, dst3, ones_blk, zeros_blk)


def _sc_agg(h, src3, dst3, zeros_blk):
    d = h.shape[1]
    cpw = src3.shape[2] // _K

    @functools.partial(
        pl.kernel,
        out_type=jax.ShapeDtypeStruct((_NC, _NPAD, d), _f32),
        mesh=_mesh(),
        scratch_types=[
            pltpu.VMEM((_K,), jnp.int32),
            pltpu.VMEM((_K,), jnp.int32),
            pltpu.VMEM((_K, d), _f32),
            pltpu.VMEM_SHARED((_NPAD, d), _f32),
            pltpu.SemaphoreType.DMA,
        ],
    )
    def agg_kernel(h_hbm, src_hbm, dst_hbm, z_hbm, out_hbm,
                   sidx, didx, rows, acc, sem):
        c = lax.axis_index("c")
        s = lax.axis_index("s")
        w = s * _NC + c
        base = pl.multiple_of(s * _RPN, 8)
        pltpu.sync_copy(z_hbm, acc.at[pl.ds(base, _RPN)])
        plsc.subcore_barrier()

        def body(j, carry):
            off = pl.multiple_of(j * _K, 8)
            pltpu.sync_copy(src_hbm.at[w, 0, pl.ds(off, _K)], sidx)
            pltpu.sync_copy(dst_hbm.at[w, 0, pl.ds(off, _K)], didx)
            pltpu.async_copy(h_hbm.at[sidx], rows, sem).wait()
            pltpu.sync_copy(rows, acc.at[didx], add=True)
            return carry

        lax.fori_loop(0, cpw, body, 0)
        plsc.subcore_barrier()
        pltpu.sync_copy(acc.at[pl.ds(base, _RPN)],
                        out_hbm.at[c, pl.ds(base, _RPN)])

    return agg_kernel(h, src3, dst3, zeros_blk)


def _sc_agg_dual(hn, hg, src3, dst3, zeros_blk):
    d = hn.shape[1]
    cpw = src3.shape[2] // _K

    @functools.partial(
        pl.kernel,
        out_type=(
            jax.ShapeDtypeStruct((_NPAD, d), _f32),
            jax.ShapeDtypeStruct((_NPAD, d), _f32),
        ),
        mesh=_mesh(),
        scratch_types=[
            pltpu.VMEM((_K,), jnp.int32),
            pltpu.VMEM((_K,), jnp.int32),
            pltpu.VMEM((_K, d), _f32),
            pltpu.VMEM_SHARED((_NPAD, d), _f32),
            pltpu.SemaphoreType.DMA,
        ],
    )
    def dual_kernel(hn_hbm, hg_hbm, src_hbm, dst_hbm, z_hbm, outn_hbm, outg_hbm,
                    sidx, didx, rows, acc, sem):
        c = lax.axis_index("c")
        s = lax.axis_index("s")
        base = pl.multiple_of(s * _RPN, 8)
        pltpu.sync_copy(z_hbm, acc.at[pl.ds(base, _RPN)])
        plsc.subcore_barrier()

        def run(h_hbm):
            def outer(b, carry):
                blk = 2 * s + b

                def body(j, carry2):
                    off = pl.multiple_of(j * _K, 8)
                    pltpu.sync_copy(src_hbm.at[blk, 0, pl.ds(off, _K)], sidx)
                    pltpu.sync_copy(dst_hbm.at[blk, 0, pl.ds(off, _K)], didx)
                    pltpu.async_copy(h_hbm.at[sidx], rows, sem).wait()
                    pltpu.sync_copy(rows, acc.at[didx], add=True)
                    return carry2

                lax.fori_loop(0, cpw, body, 0)
                return carry

            lax.fori_loop(0, 2, outer, 0)

        @pl.when(c == 0)
        def _():
            run(hn_hbm)

        @pl.when(c == 1)
        def _():
            run(hg_hbm)

        plsc.subcore_barrier()

        @pl.when(c == 0)
        def _():
            pltpu.sync_copy(acc.at[pl.ds(base, _RPN)],
                            outn_hbm.at[pl.ds(base, _RPN)])

        @pl.when(c == 1)
        def _():
            pltpu.sync_copy(acc.at[pl.ds(base, _RPN)],
                            outg_hbm.at[pl.ds(base, _RPN)])

    return dual_kernel(hn, hg, src3, dst3, zeros_blk)


_B = 2000


def _norms(degs_ref, degd_ref):
    ns = 1.0 / jnp.sqrt(jnp.maximum(degs_ref[:, 0:1], 1.0))
    nd = 1.0 / jnp.sqrt(jnp.maximum(degd_ref[:, 0:1], 1.0))
    return ns, nd


def _dot(a, b):
    return lax.dot_general(a, b, (((1,), (0,)), ((), ())),
                           preferred_element_type=_f32)


def _row_spec(d):
    return pl.BlockSpec((_B, d), lambda i: (i, 0))


def _agg_spec(d):
    return pl.BlockSpec((_NC, _B, d), lambda i: (0, i, 0))


def _const_spec(shape):
    nd = len(shape)
    return pl.BlockSpec(shape, lambda i, _nd=nd: (0,) * _nd)


def _tc_prescale(x, degs, degd):
    n, d = x.shape

    def body(x_ref, degs_ref, degd_ref, o_ref):
        ns, _ = _norms(degs_ref, degd_ref)
        o_ref[...] = x_ref[...] * ns

    return pl.pallas_call(
        body,
        grid=(n // _B,),
        in_specs=[_row_spec(d), _row_spec(d), _row_spec(d)],
        out_specs=_row_spec(d),
        out_shape=jax.ShapeDtypeStruct((n, d), _f32))(x, degs, degd)


def _tc_layer1(aggp, degs, degd, W1, n):
    d = aggp.shape[2]

    def body(a_ref, degs_ref, degd_ref, w_ref, o_ref):
        ns, nd = _norms(degs_ref, degd_ref)
        a = (a_ref[0] + a_ref[1]) * nd
        o_ref[...] = jnp.maximum(_dot(a, w_ref[...]), 0.0) * ns

    return pl.pallas_call(
        body,
        grid=(n // _B,),
        in_specs=[_agg_spec(d), _row_spec(d), _row_spec(d),
                  _const_spec((d, d))],
        out_specs=_row_spec(d),
        out_shape=jax.ShapeDtypeStruct((n, d), _f32))(aggp, degs, degd, W1)


def _tc_layer2(aggp, degs, degd, W2, Wa, ba, n):
    d = aggp.shape[2]

    def body(a_ref, degs_ref, degd_ref, w2_ref, wa_ref, ba_ref,
             adapted_ref, t_ref):
        ns, nd = _norms(degs_ref, degd_ref)
        a = (a_ref[0] + a_ref[1]) * nd
        w2a = _dot(w2_ref[...], wa_ref[...])
        adapted = _dot(a, w2a) + ba_ref[...]
        adapted_ref[...] = adapted
        t_ref[...] = adapted * ns

    return pl.pallas_call(
        body,
        grid=(n // _B,),
        in_specs=[_agg_spec(d), _row_spec(d), _row_spec(d),
                  _const_spec((d, d)), _const_spec((d, d)),
                  _const_spec((1, d))],
        out_specs=(_row_spec(d), _row_spec(d)),
        out_shape=(
            jax.ShapeDtypeStruct((n, d), _f32),
            jax.ShapeDtypeStruct((n, d), _f32),
        ))(aggp, degs, degd, W2, Wa, ba)


def _tc_layer3(aggp, degs, degd, adapted, Wn1, Wg1, n):
    d = aggp.shape[2]

    def ln(h):
        mu = jnp.mean(h, axis=-1, keepdims=True)
        var = jnp.mean((h - mu) ** 2, axis=-1, keepdims=True)
        return (h - mu) / jnp.sqrt(var + 1e-5)

    def body(a_ref, degs_ref, degd_ref, ad_ref, wn_ref, wg_ref,
             tn_ref, tg_ref, mn_ref, mg_ref):
        i = pl.program_id(0)
        ns, nd = _norms(degs_ref, degd_ref)
        a = (a_ref[0] + a_ref[1]) * nd
        ad = ad_ref[...]
        hn1 = ln(jnp.maximum(_dot(a, wn_ref[...]), 0.0) + ad)
        hg1 = ln(jnp.maximum(_dot(a, wg_ref[...]), 0.0) + ad)
        tn_ref[...] = hn1 * ns
        tg_ref[...] = hg1 * ns
        sn = jnp.sum(hn1, axis=0, keepdims=True)
        sg = jnp.sum(hg1, axis=0, keepdims=True)

        @pl.when(i == 0)
        def _():
            mn_ref[...] = sn
            mg_ref[...] = sg

        @pl.when(i > 0)
        def _():
            mn_ref[...] += sn
            mg_ref[...] += sg

    return pl.pallas_call(
        body,
        grid=(n // _B,),
        in_specs=[_agg_spec(d), _row_spec(d), _row_spec(d),
                  _row_spec(d), _const_spec((d, d)), _const_spec((d, d))],
        out_specs=(_row_spec(d), _row_spec(d),
                   _const_spec((1, d)), _const_spec((1, d))),
        out_shape=(
            jax.ShapeDtypeStruct((n, d), _f32),
            jax.ShapeDtypeStruct((n, d), _f32),
            jax.ShapeDtypeStruct((1, d), _f32),
            jax.ShapeDtypeStruct((1, d), _f32),
        ))(aggp, degs, degd, adapted, Wn1, Wg1)


def _tc_layer4_head(aggn, aggg, degs, degd, mn1, mg1,
                    Wn2, Wg2, Wf1, bf1, Wf2, bf2, n):
    d = aggn.shape[1]
    nc = Wf2.shape[1]
    g = n // _B

    def body(an_ref, ag_ref, degs_ref, degd_ref, mn_ref, mg_ref,
             wn_ref, wg_ref, wf1_ref, bf1_ref, wf2_ref, bf2_ref, o_ref,
             accn, accg):
        i = pl.program_id(0)
        _, nd = _norms(degs_ref, degd_ref)
        sn = jnp.sum(an_ref[...] * nd, axis=0, keepdims=True)
        sg = jnp.sum(ag_ref[...] * nd, axis=0, keepdims=True)

        @pl.when(i == 0)
        def _():
            accn[...] = sn
            accg[...] = sg

        @pl.when(i > 0)
        def _():
            accn[...] += sn
            accg[...] += sg

        @pl.when(i == g - 1)
        def _():
            inv = 1.0 / n
            rep_n = _dot(accn[...] * inv, wn_ref[...]) + mn_ref[...] * inv
            rep_g = _dot(accg[...] * inv, wg_ref[...]) + mg_ref[...] * inv
            fused = jnp.concatenate([rep_n, rep_g], axis=-1)
            h = jnp.maximum(_dot(fused, wf1_ref[...]) + bf1_ref[...], 0.0)
            o_ref[...] = _dot(h, wf2_ref[...]) + bf2_ref[...]

    return pl.pallas_call(
        body,
        grid=(g,),
        in_specs=[pl.BlockSpec((_B, d), lambda i: (i, 0)),
                  pl.BlockSpec((_B, d), lambda i: (i, 0)),
                  _row_spec(d), _row_spec(d),
                  _const_spec((1, d)), _const_spec((1, d)),
                  _const_spec((d, d)), _const_spec((d, d)),
                  _const_spec((2 * d, d)), _const_spec((1, d)),
                  _const_spec((d, nc)), _const_spec((1, nc))],
        out_specs=_const_spec((1, nc)),
        out_shape=jax.ShapeDtypeStruct((1, nc), _f32),
        scratch_shapes=[pltpu.VMEM((1, d), _f32), pltpu.VMEM((1, d), _f32)],
    )(aggn, aggg, degs, degd, mn1, mg1, Wn2, Wg2, Wf1, bf1, Wf2, bf2)


def kernel(x, edge_index, W_embed1, W_embed2, W_adapt, b_adapt,
           Wn1, Wn2, Wg1, Wg2, Wf1, bf1, Wf2, bf2):
    n, d = x.shape
    e = edge_index.shape[1]
    unit = _NW * _K
    epad = ((e + unit - 1) // unit) * unit
    src_p = jnp.concatenate(
        [edge_index[0], jnp.full((epad - e,), n, jnp.int32)])
    dst_p = jnp.concatenate(
        [edge_index[1], jnp.full((epad - e,), n, jnp.int32)])
    src3 = src_p.reshape(_NW, 1, epad // _NW)
    dst3 = dst_p.reshape(_NW, 1, epad // _NW)
    ones_blk = jnp.ones((_K, d), _f32)
    zeros_blk = jnp.zeros((_RPN, d), _f32)
    ba = b_adapt.reshape(1, d)
    bf1r = bf1.reshape(1, -1)
    bf2r = bf2.reshape(1, -1)

    degs, degd = _sc_degree(src3, dst3, ones_blk, zeros_blk, d)
    t0 = _tc_prescale(x, degs, degd)
    aggp1 = _sc_agg(t0, src3, dst3, zeros_blk)
    t1 = _tc_layer1(aggp1, degs, degd, W_embed1, n)
    aggp2 = _sc_agg(t1, src3, dst3, zeros_blk)
    adapted, t2 = _tc_layer2(aggp2, degs, degd, W_embed2, W_adapt, ba, n)
    aggp3 = _sc_agg(t2, src3, dst3, zeros_blk)
    tn, tg, mn1, mg1 = _tc_layer3(aggp3, degs, degd, adapted, Wn1, Wg1, n)
    aggn4, aggg4 = _sc_agg_dual(tn, tg, src3, dst3, zeros_blk)
    logits = _tc_layer4_head(aggn4, aggg4, degs, degd, mn1, mg1,
                             Wn2, Wg2, Wf1, bf1r, Wf2, bf2r, n)
    return logits
